# baseline (speedup 1.0000x reference)
import jax
import jax.numpy as jnp
from jax import lax
from jax.experimental import pallas as pl
from jax.experimental.pallas import tpu as pltpu
from jax.experimental.pallas import tpu_sc as plsc

BATCH = 16384
X_DIM = 128
Y_EMBED_DIM = 32
Y_AMR_DIM = 16
OUT_DIM = 128
LATENT_DIM = OUT_DIM // 2
N_ROWS = 1000000

_TC_TILES = 7813
_TR = 4
_TR_STRIDE = _TC_TILES * 1024
_N_SLABS = _TR * _TC_TILES
_TQ = 601
_TBB = 13

_NC = 2
_NS = 16
_NW = _NC * _NS
_B_PER_W = BATCH // _NW


def _repack_body(in_ref, out_ref):
    out_ref[...] = jnp.swapaxes(in_ref[...].reshape(8, _TQ, 128), 0, 1)


@jax.jit
def _tc_repack(tabT):
    return pl.pallas_call(
        _repack_body,
        grid=(_TR, _TBB),
        in_specs=[pl.BlockSpec((8, _TQ * 128), lambda tr, tb: (tr, tb))],
        out_specs=pl.BlockSpec((_TQ, 8, 128), lambda tr, tb: (tr * _TBB + tb, 0, 0)),
        out_shape=jax.ShapeDtypeStruct((_N_SLABS, 8, 128), jnp.float32),
    )(tabT)


def _sc_gather_body(idx_hbm, tab_hbm, out_hbm, idx_v, bv_v, iv_all, rv, sem):
    wid = lax.axis_index("s") * _NC + lax.axis_index("c")
    base = wid * _B_PER_W
    pltpu.sync_copy(idx_hbm.at[pl.ds(base, _B_PER_W)], idx_v)
    for k in range(_B_PER_W // 16):
        sl = pl.ds(k * 16, 16)
        t = idx_v[sl]
        bv_v[sl] = ((t >> 7) << 10) + (t & 127)
    for c in range(Y_EMBED_DIM):
        off = (c // 8) * _TR_STRIDE + (c % 8) * 128
        for k in range(_B_PER_W // 16):
            sl = pl.ds(k * 16, 16)
            iv_all[c, sl] = bv_v[sl] + off
    copies = [
        pltpu.async_copy(tab_hbm.at[iv_all.at[c]], rv.at[c], sem)
        for c in range(Y_EMBED_DIM)
    ]
    for cp in copies:
        cp.wait()
    pltpu.sync_copy(rv, out_hbm.at[:, pl.ds(base, _B_PER_W)])


@jax.jit
def _sc_gather(idx, tab_lin):
    mesh = plsc.VectorSubcoreMesh(core_axis_name="c", subcore_axis_name="s")
    return pl.kernel(
        _sc_gather_body,
        out_type=jax.ShapeDtypeStruct((Y_EMBED_DIM, BATCH), jnp.float32),
        mesh=mesh,
        scratch_types=[
            pltpu.VMEM((_B_PER_W,), jnp.int32),
            pltpu.VMEM((_B_PER_W,), jnp.int32),
            pltpu.VMEM((Y_EMBED_DIM, _B_PER_W), jnp.int32),
            pltpu.VMEM((Y_EMBED_DIM, _B_PER_W), jnp.float32),
            pltpu.SemaphoreType.DMA,
        ],
        compiler_params=pltpu.CompilerParams(use_tc_tiling_on_sc=False),
    )(idx, tab_lin)


_BLK = 2048


def _enc_pre_body(x_ref, amrT_ref, wx_ref, wamr_ref, b_ref, part_ref):
    acc = jnp.dot(x_ref[...], wx_ref[...], preferred_element_type=jnp.float32)
    acc += lax.dot_general(amrT_ref[...], wamr_ref[...],
                           (((0,), (0,)), ((), ())),
                           preferred_element_type=jnp.float32)
    part_ref[...] = acc + b_ref[...]


@jax.jit
def _tc_enc_pre(x, amrT, wx, wamr, b):
    grid = (BATCH // _BLK,)
    return pl.pallas_call(
        _enc_pre_body,
        grid=grid,
        in_specs=[
            pl.BlockSpec((_BLK, X_DIM), lambda i: (i, 0)),
            pl.BlockSpec((Y_AMR_DIM, _BLK), lambda i: (0, i)),
            pl.BlockSpec((X_DIM, OUT_DIM), lambda i: (0, 0)),
            pl.BlockSpec((Y_AMR_DIM, OUT_DIM), lambda i: (0, 0)),
            pl.BlockSpec((1, OUT_DIM), lambda i: (0, 0)),
        ],
        out_specs=pl.BlockSpec((_BLK, OUT_DIM), lambda i: (i, 0)),
        out_shape=jax.ShapeDtypeStruct((BATCH, OUT_DIM), jnp.float32),
    )(x, amrT, wx, wamr, b)


def _enc_post_body(part_ref, spT_ref, wsp_ref, mu_ref, lv_ref):
    acc = part_ref[...] + lax.dot_general(
        spT_ref[...], wsp_ref[...], (((0,), (0,)), ((), ())),
        preferred_element_type=jnp.float32)
    mu_ref[...] = acc[:, :LATENT_DIM]
    lv_ref[...] = acc[:, LATENT_DIM:]


@jax.jit
def _tc_enc_post(part, spT, wsp):
    grid = (BATCH // _BLK,)
    return pl.pallas_call(
        _enc_post_body,
        grid=grid,
        in_specs=[
            pl.BlockSpec((_BLK, OUT_DIM), lambda i: (i, 0)),
            pl.BlockSpec((Y_EMBED_DIM, _BLK), lambda i: (0, i)),
            pl.BlockSpec((Y_EMBED_DIM, OUT_DIM), lambda i: (0, 0)),
        ],
        out_specs=[
            pl.BlockSpec((_BLK, LATENT_DIM), lambda i: (i, 0)),
            pl.BlockSpec((_BLK, LATENT_DIM), lambda i: (i, 0)),
        ],
        out_shape=[
            jax.ShapeDtypeStruct((BATCH, LATENT_DIM), jnp.float32),
            jax.ShapeDtypeStruct((BATCH, LATENT_DIM), jnp.float32),
        ],
    )(part, spT, wsp)


def kernel(x, y_species, y_amr, emb_table, W_enc, b_enc):
    idx = y_species.astype(jnp.int32)
    tabT = emb_table.T
    packed = _tc_repack(tabT)
    tab_lin = packed.reshape(_N_SLABS * 1024)
    spT = _sc_gather(idx, tab_lin)
    wx = W_enc[:X_DIM]
    wsp = W_enc[X_DIM:X_DIM + Y_EMBED_DIM]
    wamr = W_enc[X_DIM + Y_EMBED_DIM:]
    part = _tc_enc_pre(x, y_amr.T, wx, wamr, b_enc.reshape(1, OUT_DIM))
    mu, lv = _tc_enc_post(part, spT, wsp)
    return mu, lv


# speedup vs baseline: 1.1198x; 1.1198x over previous
import jax
import jax.numpy as jnp
from jax import lax
from jax.experimental import pallas as pl
from jax.experimental.pallas import tpu as pltpu
from jax.experimental.pallas import tpu_sc as plsc

BATCH = 16384
X_DIM = 128
Y_EMBED_DIM = 32
Y_AMR_DIM = 16
OUT_DIM = 128
LATENT_DIM = OUT_DIM // 2
N_ROWS = 1000000

_TC_TILES = 7813
_TR = 4
_TR_STRIDE = _TC_TILES * 1024
_N_SLABS = _TR * _TC_TILES
_TQ = 601
_TBB = 13

_NC = 2
_NS = 16
_NW = _NC * _NS
_B_PER_W = BATCH // _NW


def _repack_body(in_ref, out_ref):
    out_ref[...] = jnp.swapaxes(in_ref[...].reshape(8, _TQ, 128), 0, 1)


@jax.jit
def _tc_repack(tabT):
    return pl.pallas_call(
        _repack_body,
        grid=(_TR, _TBB),
        in_specs=[pl.BlockSpec((8, _TQ * 128), lambda tr, tb: (tr, tb))],
        out_specs=pl.BlockSpec((_TQ, 8, 128), lambda tr, tb: (tr * _TBB + tb, 0, 0)),
        out_shape=jax.ShapeDtypeStruct((_N_SLABS, 8, 128), jnp.float32),
    )(tabT)


def _sc_gather_body(idx_hbm, tab_hbm, out_hbm, idx_v, bv_v, iv_all, rv, sem):
    wid = lax.axis_index("s") * _NC + lax.axis_index("c")
    base = wid * _B_PER_W
    pltpu.sync_copy(idx_hbm.at[pl.ds(base, _B_PER_W)], idx_v)
    for k in range(_B_PER_W // 16):
        sl = pl.ds(k * 16, 16)
        t = idx_v[sl]
        bv_v[sl] = ((t >> 7) << 10) + (t & 127)
    for c in range(Y_EMBED_DIM):
        off = (c // 8) * _TR_STRIDE + (c % 8) * 128
        for k in range(_B_PER_W // 16):
            sl = pl.ds(k * 16, 16)
            iv_all[c, sl] = bv_v[sl] + off
    copies = [
        pltpu.async_copy(tab_hbm.at[iv_all.at[c]], rv.at[c], sem)
        for c in range(Y_EMBED_DIM)
    ]
    for cp in copies:
        cp.wait()
    pltpu.sync_copy(rv, out_hbm.at[:, pl.ds(base, _B_PER_W)])


@jax.jit
def _sc_gather(idx, tab_lin):
    mesh = plsc.VectorSubcoreMesh(core_axis_name="c", subcore_axis_name="s")
    return pl.kernel(
        _sc_gather_body,
        out_type=jax.ShapeDtypeStruct((Y_EMBED_DIM, BATCH), jnp.float32),
        mesh=mesh,
        scratch_types=[
            pltpu.VMEM((_B_PER_W,), jnp.int32),
            pltpu.VMEM((_B_PER_W,), jnp.int32),
            pltpu.VMEM((Y_EMBED_DIM, _B_PER_W), jnp.int32),
            pltpu.VMEM((Y_EMBED_DIM, _B_PER_W), jnp.float32),
            pltpu.SemaphoreType.DMA,
        ],
        compiler_params=pltpu.CompilerParams(use_tc_tiling_on_sc=False),
    )(idx, tab_lin)


_BLK = 2048


def _enc_body(x_ref, spT_ref, amrT_ref, wx_ref, wsp_ref, wamr_ref, b_ref,
              muT_ref, lvT_ref):
    accT = lax.dot_general(wx_ref[...], x_ref[...],
                           (((0,), (1,)), ((), ())),
                           preferred_element_type=jnp.float32)
    accT += lax.dot_general(wsp_ref[...], spT_ref[...],
                            (((0,), (0,)), ((), ())),
                            preferred_element_type=jnp.float32)
    accT += lax.dot_general(wamr_ref[...], amrT_ref[...],
                            (((0,), (0,)), ((), ())),
                            preferred_element_type=jnp.float32)
    accT += b_ref[...]
    muT_ref[...] = accT[:LATENT_DIM, :]
    lvT_ref[...] = accT[LATENT_DIM:, :]


@jax.jit
def _tc_encode(x, spT, amrT, wx, wsp, wamr, b):
    grid = (BATCH // _BLK,)
    return pl.pallas_call(
        _enc_body,
        grid=grid,
        in_specs=[
            pl.BlockSpec((_BLK, X_DIM), lambda i: (i, 0)),
            pl.BlockSpec((Y_EMBED_DIM, _BLK), lambda i: (0, i)),
            pl.BlockSpec((Y_AMR_DIM, _BLK), lambda i: (0, i)),
            pl.BlockSpec((X_DIM, OUT_DIM), lambda i: (0, 0)),
            pl.BlockSpec((Y_EMBED_DIM, OUT_DIM), lambda i: (0, 0)),
            pl.BlockSpec((Y_AMR_DIM, OUT_DIM), lambda i: (0, 0)),
            pl.BlockSpec((OUT_DIM, 1), lambda i: (0, 0)),
        ],
        out_specs=[
            pl.BlockSpec((LATENT_DIM, _BLK), lambda i: (0, i)),
            pl.BlockSpec((LATENT_DIM, _BLK), lambda i: (0, i)),
        ],
        out_shape=[
            jax.ShapeDtypeStruct((LATENT_DIM, BATCH), jnp.float32),
            jax.ShapeDtypeStruct((LATENT_DIM, BATCH), jnp.float32),
        ],
    )(x, spT, amrT, wx, wsp, wamr, b)


def kernel(x, y_species, y_amr, emb_table, W_enc, b_enc):
    idx = y_species.astype(jnp.int32)
    tabT = emb_table.T
    packed = _tc_repack(tabT)
    tab_lin = packed.reshape(_N_SLABS * 1024)
    spT = _sc_gather(idx, tab_lin)
    wx = W_enc[:X_DIM]
    wsp = W_enc[X_DIM:X_DIM + Y_EMBED_DIM]
    wamr = W_enc[X_DIM + Y_EMBED_DIM:]
    muT, lvT = _tc_encode(x, spT, y_amr.T, wx, wsp, wamr,
                          b_enc.reshape(OUT_DIM, 1))
    return muT.T, lvT.T


# speedup vs baseline: 1.1492x; 1.0263x over previous
import functools

import jax
import jax.numpy as jnp
from jax import lax
from jax.experimental import pallas as pl
from jax.experimental.pallas import tpu as pltpu
from jax.experimental.pallas import tpu_sc as plsc

BATCH = 16384
X_DIM = 128
Y_EMBED_DIM = 32
Y_AMR_DIM = 16
OUT_DIM = 128
LATENT_DIM = OUT_DIM // 2
N_ROWS = 1000000

_TC_TILES = 7813
_TR_STRIDE = _TC_TILES * 1024
_HALF_COLS = 16
_HALF_TR = 2
_HALF_SLABS = _HALF_TR * _TC_TILES
_TQ = 601
_TBB = 13

_NC = 2
_NS = 16
_NW = _NC * _NS
_B_PER_W = BATCH // _NW


def _repack_body(in_ref, out_ref):
    out_ref[...] = jnp.swapaxes(in_ref[...].reshape(8, _TQ, 128), 0, 1)


@functools.partial(jax.jit, static_argnums=1)
def _tc_repack_half(tabT, base_tr):
    return pl.pallas_call(
        _repack_body,
        grid=(_HALF_TR, _TBB),
        in_specs=[pl.BlockSpec((8, _TQ * 128),
                               lambda tr, tb: (base_tr + tr, tb))],
        out_specs=pl.BlockSpec((_TQ, 8, 128),
                               lambda tr, tb: (tr * _TBB + tb, 0, 0)),
        out_shape=jax.ShapeDtypeStruct((_HALF_SLABS, 8, 128), jnp.float32),
    )(tabT)


def _sc_gather_body(idx_hbm, tab_hbm, out_hbm, idx_v, bv_v, iv_all, rv, sem):
    wid = lax.axis_index("s") * _NC + lax.axis_index("c")
    base = wid * _B_PER_W
    pltpu.sync_copy(idx_hbm.at[pl.ds(base, _B_PER_W)], idx_v)
    for k in range(_B_PER_W // 16):
        sl = pl.ds(k * 16, 16)
        t = idx_v[sl]
        bv_v[sl] = ((t >> 7) << 10) + (t & 127)
    for c in range(_HALF_COLS):
        off = (c // 8) * _TR_STRIDE + (c % 8) * 128
        for k in range(_B_PER_W // 16):
            sl = pl.ds(k * 16, 16)
            iv_all[c, sl] = bv_v[sl] + off
    copies = [
        pltpu.async_copy(tab_hbm.at[iv_all.at[c]], rv.at[c], sem)
        for c in range(_HALF_COLS)
    ]
    for cp in copies:
        cp.wait()
    pltpu.sync_copy(rv, out_hbm.at[:, pl.ds(base, _B_PER_W)])


@jax.jit
def _sc_gather_half(idx, tab_lin):
    mesh = plsc.VectorSubcoreMesh(core_axis_name="c", subcore_axis_name="s")
    return pl.kernel(
        _sc_gather_body,
        out_type=jax.ShapeDtypeStruct((_HALF_COLS, BATCH), jnp.float32),
        mesh=mesh,
        scratch_types=[
            pltpu.VMEM((_B_PER_W,), jnp.int32),
            pltpu.VMEM((_B_PER_W,), jnp.int32),
            pltpu.VMEM((_HALF_COLS, _B_PER_W), jnp.int32),
            pltpu.VMEM((_HALF_COLS, _B_PER_W), jnp.float32),
            pltpu.SemaphoreType.DMA,
        ],
        compiler_params=pltpu.CompilerParams(use_tc_tiling_on_sc=False),
    )(idx, tab_lin)


_BLK = 2048


def _enc_body(x_ref, sp0T_ref, sp1T_ref, amrT_ref, wx_ref, wsp0_ref,
              wsp1_ref, wamr_ref, b_ref, muT_ref, lvT_ref):
    accT = lax.dot_general(wx_ref[...], x_ref[...],
                           (((0,), (1,)), ((), ())),
                           preferred_element_type=jnp.float32)
    accT += lax.dot_general(wsp0_ref[...], sp0T_ref[...],
                            (((0,), (0,)), ((), ())),
                            preferred_element_type=jnp.float32)
    accT += lax.dot_general(wsp1_ref[...], sp1T_ref[...],
                            (((0,), (0,)), ((), ())),
                            preferred_element_type=jnp.float32)
    accT += lax.dot_general(wamr_ref[...], amrT_ref[...],
                            (((0,), (0,)), ((), ())),
                            preferred_element_type=jnp.float32)
    accT += b_ref[...]
    muT_ref[...] = accT[:LATENT_DIM, :]
    lvT_ref[...] = accT[LATENT_DIM:, :]


@jax.jit
def _tc_encode(x, sp0T, sp1T, amrT, wx, wsp0, wsp1, wamr, b):
    grid = (BATCH // _BLK,)
    return pl.pallas_call(
        _enc_body,
        grid=grid,
        in_specs=[
            pl.BlockSpec((_BLK, X_DIM), lambda i: (i, 0)),
            pl.BlockSpec((_HALF_COLS, _BLK), lambda i: (0, i)),
            pl.BlockSpec((_HALF_COLS, _BLK), lambda i: (0, i)),
            pl.BlockSpec((Y_AMR_DIM, _BLK), lambda i: (0, i)),
            pl.BlockSpec((X_DIM, OUT_DIM), lambda i: (0, 0)),
            pl.BlockSpec((_HALF_COLS, OUT_DIM), lambda i: (0, 0)),
            pl.BlockSpec((_HALF_COLS, OUT_DIM), lambda i: (0, 0)),
            pl.BlockSpec((Y_AMR_DIM, OUT_DIM), lambda i: (0, 0)),
            pl.BlockSpec((OUT_DIM, 1), lambda i: (0, 0)),
        ],
        out_specs=[
            pl.BlockSpec((LATENT_DIM, _BLK), lambda i: (0, i)),
            pl.BlockSpec((LATENT_DIM, _BLK), lambda i: (0, i)),
        ],
        out_shape=[
            jax.ShapeDtypeStruct((LATENT_DIM, BATCH), jnp.float32),
            jax.ShapeDtypeStruct((LATENT_DIM, BATCH), jnp.float32),
        ],
    )(x, sp0T, sp1T, amrT, wx, wsp0, wsp1, wamr, b)


def kernel(x, y_species, y_amr, emb_table, W_enc, b_enc):
    idx = y_species.astype(jnp.int32)
    tabT = emb_table.T
    packed0 = _tc_repack_half(tabT, 0)
    sp0T = _sc_gather_half(idx, packed0.reshape(_HALF_SLABS * 1024))
    packed1 = _tc_repack_half(tabT, _HALF_TR)
    sp1T = _sc_gather_half(idx, packed1.reshape(_HALF_SLABS * 1024))
    wx = W_enc[:X_DIM]
    wsp0 = W_enc[X_DIM:X_DIM + _HALF_COLS]
    wsp1 = W_enc[X_DIM + _HALF_COLS:X_DIM + Y_EMBED_DIM]
    wamr = W_enc[X_DIM + Y_EMBED_DIM:]
    muT, lvT = _tc_encode(x, sp0T, sp1T, y_amr.T, wx, wsp0, wsp1, wamr,
                          b_enc.reshape(OUT_DIM, 1))
    return muT.T, lvT.T
